# Initial kernel scaffold; baseline (speedup 1.0000x reference)
import jax
import jax.numpy as jnp
from jax.experimental import pallas as pl


def kernel(x_block, x_spmt, x_crane, x_facility, ei_block_spmt, ei_spmt_block, ei_block_crane, ei_crane_block, ei_block_fac, ei_block_block, ei_spmt_fac, ei_crane_fac, batch_block, batch_spmt, batch_crane, batch_facility, params):
    raise NotImplementedError("write your pallas kernel here")


# speedup vs baseline: 1.0690x; 1.0690x over previous
import functools

import jax
import jax.numpy as jnp
from jax.experimental import pallas as pl
from jax.experimental.pallas import tpu as pltpu

NB, NS, NC, NF = 100000, 5000, 2000, 200
HID, HEADS, NG = 128, 4, 64
OUTC = HID // HEADS
NNODES = {'block': NB, 'spmt': NS, 'crane': NC, 'facility': NF}
NODE_ORDER = ['block', 'spmt', 'crane', 'facility']
EDGE_DEFS = [
    ('block', 'spmt', 'ei_block_spmt', 100000),
    ('spmt', 'block', 'ei_spmt_block', 100000),
    ('block', 'crane', 'ei_block_crane', 100000),
    ('crane', 'block', 'ei_crane_block', 100000),
    ('block', 'facility', 'ei_block_fac', 100000),
    ('block', 'block', 'ei_block_block', 100000),
    ('spmt', 'facility', 'ei_spmt_fac', 5000),
    ('crane', 'facility', 'ei_crane_fac', 2000),
]


def _matmul_body(x_ref, w_ref, o_ref):
    o_ref[...] = jnp.dot(x_ref[...], w_ref[...],
                         preferred_element_type=jnp.float32)


def _pallas_matmul(x, w, row_block):
    n, k = x.shape
    _, m = w.shape
    grid = (n // row_block,)
    return pl.pallas_call(
        _matmul_body,
        grid=grid,
        in_specs=[
            pl.BlockSpec((row_block, k), lambda i: (i, 0)),
            pl.BlockSpec((k, m), lambda i: (0, 0)),
        ],
        out_specs=pl.BlockSpec((row_block, m), lambda i: (i, 0)),
        out_shape=jax.ShapeDtypeStruct((n, m), jnp.float32),
    )(x, w)


def _gat(x_src, x_dst, ei, p, n_dst):
    h_src = (x_src @ p['W']).reshape(-1, HEADS, OUTC)
    a_src = (h_src * p['att_src']).sum(-1)
    a_dst = (x_dst @ p['W']).reshape(-1, HEADS, OUTC)
    a_dst = (a_dst * p['att_dst']).sum(-1)
    s, d = ei[0], ei[1]
    alpha = jax.nn.leaky_relu(a_src[s] + a_dst[d], negative_slope=0.2)
    e = jnp.exp(alpha)
    denom = jax.ops.segment_sum(e, d, num_segments=n_dst)
    acc = jax.ops.segment_sum(h_src[s] * e[:, :, None], d, num_segments=n_dst)
    out = acc / (denom[:, :, None] + 1e-16)
    return out.reshape(n_dst, HID) + p['bias']


def kernel(x_block, x_spmt, x_crane, x_facility,
           ei_block_spmt, ei_spmt_block, ei_block_crane, ei_crane_block,
           ei_block_fac, ei_block_block, ei_spmt_fac, ei_crane_fac,
           batch_block, batch_spmt, batch_crane, batch_facility, params):
    xs = {'block': x_block, 'spmt': x_spmt, 'crane': x_crane,
          'facility': x_facility}
    edges = {'ei_block_spmt': ei_block_spmt, 'ei_spmt_block': ei_spmt_block,
             'ei_block_crane': ei_block_crane, 'ei_crane_block': ei_crane_block,
             'ei_block_fac': ei_block_fac, 'ei_block_block': ei_block_block,
             'ei_spmt_fac': ei_spmt_fac, 'ei_crane_fac': ei_crane_fac}
    batches = {'block': batch_block, 'spmt': batch_spmt, 'crane': batch_crane,
               'facility': batch_facility}

    x = {}
    for nt in NODE_ORDER:
        xi = xs[nt]
        w = params['proj'][nt]['W']
        kdim = xi.shape[1]
        kp = 16
        xi_p = jnp.pad(xi, ((0, 0), (0, kp - kdim)))
        w_p = jnp.pad(w, ((0, kp - kdim), (0, 0)))
        n = xi.shape[0]
        rb = {100000: 1000, 5000: 1000, 2000: 1000, 200: 200}[n]
        x[nt] = _pallas_matmul(xi_p, w_p, rb) + params['proj'][nt]['b']

    for rel, norm in zip(params['layers'], params['norms']):
        sums = {nt: jnp.zeros((NNODES[nt], HID), jnp.float32)
                for nt in NODE_ORDER}
        cnt = {nt: 0 for nt in NODE_ORDER}
        for s, d, name, E in EDGE_DEFS:
            sums[d] = sums[d] + _gat(x[s], x[d], edges[name], rel[name],
                                     NNODES[d])
            cnt[d] += 1
        newx = {}
        for nt in NODE_ORDER:
            v = sums[nt] / cnt[nt]
            v = jax.nn.relu(v) + x[nt]
            mu = v.mean(-1, keepdims=True)
            var = ((v - mu) ** 2).mean(-1, keepdims=True)
            newx[nt] = (v - mu) / jnp.sqrt(var + 1e-5) * norm['gamma'] \
                + norm['beta']
        x = newx

    pooled = []
    for nt in NODE_ORDER:
        ssum = jax.ops.segment_sum(x[nt], batches[nt], num_segments=NG)
        c = jax.ops.segment_sum(jnp.ones((NNODES[nt], 1), jnp.float32),
                                batches[nt], num_segments=NG)
        pooled.append(ssum / jnp.maximum(c, 1.0))
    return jnp.concatenate(pooled, axis=-1)


# speedup vs baseline: 1.0722x; 1.0029x over previous
import functools

import jax
import jax.numpy as jnp
from jax import lax
from jax.experimental import pallas as pl
from jax.experimental.pallas import tpu as pltpu
from jax.experimental.pallas import tpu_sc as plsc

SC_CORES, SC_SUBCORES, LANES = 2, 16, 16
N_TILES = SC_CORES * SC_SUBCORES

NB, NS, NC, NF = 100000, 5000, 2000, 200
HID, HEADS, NG = 128, 4, 64
OUTC = HID // HEADS
NNODES = {'block': NB, 'spmt': NS, 'crane': NC, 'facility': NF}
NODE_ORDER = ['block', 'spmt', 'crane', 'facility']
EDGE_DEFS = [
    ('block', 'spmt', 'ei_block_spmt', 100000),
    ('spmt', 'block', 'ei_spmt_block', 100000),
    ('block', 'crane', 'ei_block_crane', 100000),
    ('crane', 'block', 'ei_crane_block', 100000),
    ('block', 'facility', 'ei_block_fac', 100000),
    ('block', 'block', 'ei_block_block', 100000),
    ('spmt', 'facility', 'ei_spmt_fac', 5000),
    ('crane', 'facility', 'ei_crane_fac', 2000),
]

_MESH = plsc.VectorSubcoreMesh(core_axis_name="c", subcore_axis_name="s",
                               num_cores=SC_CORES, num_subcores=SC_SUBCORES)


def _matmul_body(x_ref, w_ref, o_ref):
    o_ref[...] = jnp.dot(x_ref[...], w_ref[...],
                         preferred_element_type=jnp.float32)


def _pallas_matmul(x, w, row_block):
    n, k = x.shape
    _, m = w.shape
    return pl.pallas_call(
        _matmul_body,
        grid=(n // row_block,),
        in_specs=[
            pl.BlockSpec((row_block, k), lambda i: (i, 0)),
            pl.BlockSpec((k, m), lambda i: (0, 0)),
        ],
        out_specs=pl.BlockSpec((row_block, m), lambda i: (i, 0)),
        out_shape=jax.ShapeDtypeStruct((n, m), jnp.float32),
    )(x, w)


def _fill_zeros(ref, nwords):
    z = jnp.zeros((LANES,), jnp.float32)

    @pl.loop(0, nwords // LANES)
    def _(i):
        ref[pl.ds(i * LANES, LANES)] = z


def _fill_zeros2d(ref, rows, cols):
    z = jnp.zeros((LANES,), jnp.float32)
    cv = cols // LANES

    @pl.loop(0, rows * cv)
    def _(i):
        ref[i // cv, pl.ds((i % cv) * LANES, LANES)] = z


def _fill_ones(ref, nwords):
    o = jnp.ones((LANES,), jnp.float32)

    @pl.loop(0, nwords // LANES)
    def _(i):
        ref[pl.ds(i * LANES, LANES)] = o


@functools.partial(jax.jit, static_argnames=("npad", "bsz"))
def _pool_sc(xp, bp, npad, bsz):
    T = npad // N_TILES
    iters = T // bsz

    @functools.partial(
        pl.kernel, mesh=_MESH,
        out_type=(jax.ShapeDtypeStruct((SC_CORES, NG, HID), jnp.float32),
                  jax.ShapeDtypeStruct((SC_CORES, 128), jnp.float32)),
        scratch_types=[
            pltpu.VMEM((bsz, HID), jnp.float32),
            pltpu.VMEM((bsz,), jnp.int32),
            pltpu.VMEM((bsz,), jnp.float32),
            pltpu.VMEM((NG + 16, HID), jnp.float32),
            pltpu.VMEM((128,), jnp.float32),
            pltpu.VMEM_SHARED((NG + 16, HID), jnp.float32),
            pltpu.VMEM_SHARED((128,), jnp.float32),
        ],
    )
    def pool(x_hbm, b_hbm, sum_out, cnt_out, rows_v, idx_v, ones_v, zacc_v,
             zcnt_v, acc_sh, cnt_sh):
        c = lax.axis_index("c")
        s = lax.axis_index("s")
        tile = c * SC_SUBCORES + s
        _fill_ones(ones_v, bsz)

        @pl.when(s == 0)
        def _():
            _fill_zeros2d(zacc_v, NG + 16, HID)
            _fill_zeros(zcnt_v, 128)
            pltpu.sync_copy(zacc_v, acc_sh)
            pltpu.sync_copy(zcnt_v, cnt_sh)

        plsc.subcore_barrier()
        base = tile * T

        @pl.loop(0, iters)
        def _(i):
            off = base + i * bsz
            pltpu.sync_copy(b_hbm.at[pl.ds(off, bsz)], idx_v)
            pltpu.sync_copy(x_hbm.at[pl.ds(off, bsz)], rows_v)
            pltpu.sync_copy(rows_v, acc_sh.at[idx_v], add=True)
            pltpu.sync_copy(ones_v, cnt_sh.at[idx_v], add=True)

        plsc.subcore_barrier()

        @pl.when(s == 0)
        def _():
            pltpu.sync_copy(acc_sh.at[pl.ds(0, NG)], sum_out.at[c])
            pltpu.sync_copy(cnt_sh, cnt_out.at[c])

    return pool(xp, bp)


def _pool(x, batch, n):
    npad = {NB: 100352, NS: 5120, NC: 2048, NF: 512}[n]
    bsz = {NB: 448, NS: 160, NC: 64, NF: 16}[n]
    xp = jnp.pad(x, ((0, npad - n), (0, 0)))
    bp = jnp.pad(batch.astype(jnp.int32), (0, npad - n), constant_values=NG)
    sums, cnts = _pool_sc(xp, bp, npad, bsz)
    ssum = sums[0] + sums[1]
    cc = cnts[0, :NG] + cnts[1, :NG]
    return ssum / jnp.maximum(cc, 1.0)[:, None]


def _gat(x_src, x_dst, ei, p, n_dst):
    h_src = (x_src @ p['W']).reshape(-1, HEADS, OUTC)
    a_src = (h_src * p['att_src']).sum(-1)
    a_dst = (x_dst @ p['W']).reshape(-1, HEADS, OUTC)
    a_dst = (a_dst * p['att_dst']).sum(-1)
    s, d = ei[0], ei[1]
    alpha = jax.nn.leaky_relu(a_src[s] + a_dst[d], negative_slope=0.2)
    e = jnp.exp(alpha)
    denom = jax.ops.segment_sum(e, d, num_segments=n_dst)
    acc = jax.ops.segment_sum(h_src[s] * e[:, :, None], d, num_segments=n_dst)
    out = acc / (denom[:, :, None] + 1e-16)
    return out.reshape(n_dst, HID) + p['bias']


def kernel(x_block, x_spmt, x_crane, x_facility,
           ei_block_spmt, ei_spmt_block, ei_block_crane, ei_crane_block,
           ei_block_fac, ei_block_block, ei_spmt_fac, ei_crane_fac,
           batch_block, batch_spmt, batch_crane, batch_facility, params):
    xs = {'block': x_block, 'spmt': x_spmt, 'crane': x_crane,
          'facility': x_facility}
    edges = {'ei_block_spmt': ei_block_spmt, 'ei_spmt_block': ei_spmt_block,
             'ei_block_crane': ei_block_crane, 'ei_crane_block': ei_crane_block,
             'ei_block_fac': ei_block_fac, 'ei_block_block': ei_block_block,
             'ei_spmt_fac': ei_spmt_fac, 'ei_crane_fac': ei_crane_fac}
    batches = {'block': batch_block, 'spmt': batch_spmt, 'crane': batch_crane,
               'facility': batch_facility}

    x = {}
    for nt in NODE_ORDER:
        xi = xs[nt]
        w = params['proj'][nt]['W']
        kdim = xi.shape[1]
        xi_p = jnp.pad(xi, ((0, 0), (0, 16 - kdim)))
        w_p = jnp.pad(w, ((0, 16 - kdim), (0, 0)))
        rb = 200 if nt == 'facility' else 1000
        x[nt] = _pallas_matmul(xi_p, w_p, rb) + params['proj'][nt]['b']

    for rel, norm in zip(params['layers'], params['norms']):
        sums = {nt: jnp.zeros((NNODES[nt], HID), jnp.float32)
                for nt in NODE_ORDER}
        cnt = {nt: 0 for nt in NODE_ORDER}
        for s, d, name, E in EDGE_DEFS:
            sums[d] = sums[d] + _gat(x[s], x[d], edges[name], rel[name],
                                     NNODES[d])
            cnt[d] += 1
        newx = {}
        for nt in NODE_ORDER:
            v = sums[nt] / cnt[nt]
            v = jax.nn.relu(v) + x[nt]
            mu = v.mean(-1, keepdims=True)
            var = ((v - mu) ** 2).mean(-1, keepdims=True)
            newx[nt] = (v - mu) / jnp.sqrt(var + 1e-5) * norm['gamma'] \
                + norm['beta']
        x = newx

    pooled = [_pool(x[nt], batches[nt], NNODES[nt]) for nt in NODE_ORDER]
    return jnp.concatenate(pooled, axis=-1)
